# baseline (speedup 1.0000x reference)
import functools

import jax
import jax.numpy as jnp
from jax import lax
from jax.experimental import pallas as pl
from jax.experimental.pallas import tpu as pltpu
from jax.experimental.pallas import tpu_sc as plsc

_NC = 2
_NS = 16
_NW = _NC * _NS

_HID = 64
_OTHER = 32
_NF = 101
_B = 1024
_L = 100
_HLEN = 50
_FEAT_ONE = _NF * 2 * _HID
_DIM = _FEAT_ONE + 8 * _OTHER
_FROWS = 104
_LP = 104
_HLP = 56
_RPW = _B // _NW


def _sc_feat_kernel(indexes, values, fields, uid, did, history, ti, wk, ts,
                    pid, cold, emb_table, field_table, user_table, doc_table,
                    time_table, weekday_table, timespan_table, product_table,
                    cold_table, feat,
                    idx_v, fld_v, val_v, hist_v, uid_v, did_v, ti_v, wk_v,
                    ts_v, pid_v, cold_v, emb_rows, ftab_v, pooled, hist_rows,
                    u_rows, d_rows, t_rows, w_rows, s_rows, p_rows, c_rows,
                    sem, sem_g0, sem_g1, sem_w0, sem_w1):
    sems_g = (sem_g0, sem_g1)
    sems_w = (sem_w0, sem_w1)
    wid = lax.axis_index("s") * _NC + lax.axis_index("c")
    base = wid * _RPW

    pltpu.sync_copy(indexes.at[pl.ds(base * _LP, _RPW * _LP)], idx_v)
    pltpu.sync_copy(fields.at[pl.ds(base * _LP, _RPW * _LP)],
                    fld_v.at[pl.ds(0, _RPW * _LP)])
    pltpu.sync_copy(values.at[pl.ds(base * _LP, _RPW * _LP)],
                    val_v.at[pl.ds(0, _RPW * _LP)])
    pltpu.sync_copy(history.at[pl.ds(base * _HLP, _RPW * _HLP)],
                    hist_v.at[pl.ds(0, _RPW * _HLP)])
    sl = pl.ds(base, _RPW)
    pltpu.sync_copy(uid.at[sl], uid_v)
    pltpu.sync_copy(did.at[sl], did_v)
    pltpu.sync_copy(ti.at[sl], ti_v)
    pltpu.sync_copy(wk.at[sl], wk_v)
    pltpu.sync_copy(ts.at[sl], ts_v)
    pltpu.sync_copy(pid.at[sl], pid_v)
    pltpu.sync_copy(cold.at[sl], cold_v)
    pltpu.sync_copy(field_table, ftab_v)

    pltpu.async_copy(user_table.at[uid_v], u_rows, sem).wait()
    pltpu.async_copy(doc_table.at[did_v], d_rows, sem).wait()
    pltpu.async_copy(time_table.at[ti_v], t_rows, sem).wait()
    pltpu.async_copy(weekday_table.at[wk_v], w_rows, sem).wait()
    pltpu.async_copy(timespan_table.at[ts_v], s_rows, sem).wait()
    pltpu.async_copy(product_table.at[pid_v], p_rows, sem).wait()
    pltpu.async_copy(cold_table.at[cold_v], c_rows, sem).wait()

    def fire_gathers(r, p):
        pltpu.async_copy(emb_table.at[idx_v.at[pl.ds(r * _LP, _L)]],
                         emb_rows.at[p], sems_g[p])
        pltpu.async_copy(doc_table.at[hist_v.at[pl.ds(r * _HLP, _HLEN)]],
                         hist_rows.at[p], sems_g[p])

    def compute_row(r, p, do_wait_wb):
        @pl.when(do_wait_wb)
        def _wait_wb():
            pltpu.make_async_copy(pooled.at[p], feat.at[base + r - 2],
                                  sems_w[p]).wait()

        def zero_body(i, c):
            for cc in range(8):
                pooled[p, i, pl.ds(cc * 16, 16)] = jnp.zeros((16,),
                                                             jnp.float32)
            return c
        lax.fori_loop(0, _NF, zero_body, 0)

        pltpu.make_async_copy(emb_table.at[idx_v.at[pl.ds(r * _LP, _L)]],
                              emb_rows.at[p], sems_g[p]).wait()
        pltpu.make_async_copy(doc_table.at[hist_v.at[pl.ds(r * _HLP, _HLEN)]],
                              hist_rows.at[p], sems_g[p]).wait()

        def acc_lanes(lc, nlanes):
            vv = val_v[pl.ds(r * _LP + lc * 16, 16)]
            fv = fld_v[pl.ds(r * _LP + lc * 16, 16)]
            for j in range(nlanes):
                v = vv[j]
                f = fv[j]
                l = lc * 16 + j
                for cc in range(_HID // 16):
                    e = emb_rows[p, l, pl.ds(cc * 16, 16)]
                    plsc.addupdate(pooled.at[p, f, pl.ds(cc * 16, 16)], e * v)
                for cc in range(_HID // 16):
                    ft = ftab_v[pl.ds(f * _HID + cc * 16, 16)]
                    plsc.addupdate(pooled.at[p, f, pl.ds(_HID + cc * 16, 16)],
                                   ft * v)

        def acc_body(lc, c):
            acc_lanes(lc, 16)
            return c
        lax.fori_loop(0, _L // 16, acc_body, 0)
        acc_lanes(_L // 16, _L % 16)

        z16 = jnp.zeros((16,), jnp.float32)

        def hist_lanes(lc, nlanes, carry):
            s0, s1, cnt = carry
            hv = hist_v[pl.ds(r * _HLP + lc * 16, 16)]
            for j in range(nlanes):
                l = lc * 16 + j
                m = jnp.where(hv[j] != 0, jnp.float32(1.0), jnp.float32(0.0))
                s0 = s0 + hist_rows[p, l, pl.ds(0, 16)] * m
                s1 = s1 + hist_rows[p, l, pl.ds(16, 16)] * m
                cnt = cnt + m
            return (s0, s1, cnt)

        def hist_body(lc, carry):
            return hist_lanes(lc, 16, carry)
        carry = lax.fori_loop(0, _HLEN // 16, hist_body,
                              (z16, z16, jnp.float32(0.0)))
        s0, s1, cnt = hist_lanes(_HLEN // 16, _HLEN % 16, carry)
        inv = (jnp.ones((16,), jnp.float32)
               / jnp.full((16,), jnp.maximum(cnt, 1.0)))

        pooled[p, _NF, pl.ds(64, 16)] = s0 * inv
        pooled[p, _NF, pl.ds(80, 16)] = s1 * inv
        for buf, row, off in ((u_rows, _NF, 0), (d_rows, _NF, 32),
                              (t_rows, _NF, 96), (w_rows, _NF + 1, 0),
                              (s_rows, _NF + 1, 32), (p_rows, _NF + 1, 64),
                              (c_rows, _NF + 1, 96)):
            pooled[p, row, pl.ds(off, 16)] = buf[r, pl.ds(0, 16)]
            pooled[p, row, pl.ds(off + 16, 16)] = buf[r, pl.ds(16, 16)]

        pltpu.async_copy(pooled.at[p], feat.at[base + r], sems_w[p])

    fire_gathers(0, 0)

    def pair_body(g, c):
        fire_gathers(2 * g + 1, 1)
        compute_row(2 * g, 0, g > 0)

        @pl.when(g < _RPW // 2 - 1)
        def _prefetch_even():
            fire_gathers(2 * g + 2, 0)

        compute_row(2 * g + 1, 1, g > 0)
        return c

    lax.fori_loop(0, _RPW // 2, pair_body, 0)
    pltpu.make_async_copy(pooled.at[0], feat.at[base + _RPW - 2],
                          sems_w[0]).wait()
    pltpu.make_async_copy(pooled.at[1], feat.at[base + _RPW - 1],
                          sems_w[1]).wait()


def _make_sc_feat():
    mesh = plsc.VectorSubcoreMesh(core_axis_name="c", subcore_axis_name="s")
    return functools.partial(
        pl.kernel, mesh=mesh,
        compiler_params=pltpu.CompilerParams(use_tc_tiling_on_sc=False),
        out_type=jax.ShapeDtypeStruct((_B, _FROWS, 128), jnp.float32),
        scratch_types=[
            pltpu.VMEM((_RPW * _LP,), jnp.int32),
            pltpu.VMEM((_RPW * _LP + 16,), jnp.int32),
            pltpu.VMEM((_RPW * _LP + 16,), jnp.float32),
            pltpu.VMEM((_RPW * _HLP + 16,), jnp.int32),
            pltpu.VMEM((_RPW,), jnp.int32),
            pltpu.VMEM((_RPW,), jnp.int32),
            pltpu.VMEM((_RPW,), jnp.int32),
            pltpu.VMEM((_RPW,), jnp.int32),
            pltpu.VMEM((_RPW,), jnp.int32),
            pltpu.VMEM((_RPW,), jnp.int32),
            pltpu.VMEM((_RPW,), jnp.int32),
            pltpu.VMEM((2, _L, _HID), jnp.float32),
            pltpu.VMEM((_NF * _HID,), jnp.float32),
            pltpu.VMEM((2, _FROWS, 128), jnp.float32),
            pltpu.VMEM((2, _HLEN, _OTHER), jnp.float32),
            pltpu.VMEM((_RPW, _OTHER), jnp.float32),
            pltpu.VMEM((_RPW, _OTHER), jnp.float32),
            pltpu.VMEM((_RPW, _OTHER), jnp.float32),
            pltpu.VMEM((_RPW, _OTHER), jnp.float32),
            pltpu.VMEM((_RPW, _OTHER), jnp.float32),
            pltpu.VMEM((_RPW, _OTHER), jnp.float32),
            pltpu.VMEM((_RPW, _OTHER), jnp.float32),
            pltpu.SemaphoreType.DMA,
            pltpu.SemaphoreType.DMA,
            pltpu.SemaphoreType.DMA,
            pltpu.SemaphoreType.DMA,
            pltpu.SemaphoreType.DMA,
        ])(_sc_feat_kernel)


def _mlp_body(feat_ref, w1_ref, b1_ref, w2_ref, b2_ref, wd_ref, bd_ref,
              out_ref):
    x = feat_ref[...].astype(jnp.bfloat16)
    h = jnp.dot(x, w1_ref[...], preferred_element_type=jnp.float32)
    h = jnp.maximum(h + b1_ref[...], 0.0)
    h = jnp.dot(h.astype(jnp.bfloat16), w2_ref[...],
                preferred_element_type=jnp.float32)
    h = jnp.maximum(h + b2_ref[...], 0.0)
    out_ref[...] = (jnp.dot(h, wd_ref[...], preferred_element_type=jnp.float32)
                    + bd_ref[...])


def _mlp(feat, W1, b1, W2, b2, Wd, bd):
    mb = 128
    grid = (_B // mb,)
    return pl.pallas_call(
        _mlp_body,
        grid=grid,
        in_specs=[
            pl.BlockSpec((mb, _DIM), lambda i: (i, 0)),
            pl.BlockSpec((_DIM, W1.shape[1]), lambda i: (0, 0)),
            pl.BlockSpec((1, W1.shape[1]), lambda i: (0, 0)),
            pl.BlockSpec((W2.shape[0], W2.shape[1]), lambda i: (0, 0)),
            pl.BlockSpec((1, W2.shape[1]), lambda i: (0, 0)),
            pl.BlockSpec((Wd.shape[0], 1), lambda i: (0, 0)),
            pl.BlockSpec((1, 1), lambda i: (0, 0)),
        ],
        out_specs=pl.BlockSpec((mb, 1), lambda i: (i, 0)),
        out_shape=jax.ShapeDtypeStruct((_B, 1), jnp.float32),
        compiler_params=pltpu.CompilerParams(
            dimension_semantics=("arbitrary",)),
    )(feat, W1.astype(jnp.bfloat16), b1.reshape(1, -1),
      W2.astype(jnp.bfloat16), b2.reshape(1, -1), Wd, bd.reshape(1, 1))


def kernel(indexes, values, fields, uid, did, history, time_interval,
           time_weekday, timespan_interval, product_id, cold, emb_table,
           field_table, user_table, doc_table, time_table, weekday_table,
           timespan_table, product_table, cold_table, W1, b1, W2, b2, Wd, bd):
    sc_feat = _make_sc_feat()
    pad_l = ((0, 0), (0, _LP - _L))
    pad_h = ((0, 0), (0, _HLP - _HLEN))
    feat = sc_feat(jnp.pad(indexes.astype(jnp.int32), pad_l).reshape(-1),
                   jnp.pad(values, pad_l).reshape(-1),
                   jnp.pad(fields.astype(jnp.int32), pad_l).reshape(-1),
                   uid.astype(jnp.int32), did.astype(jnp.int32),
                   jnp.pad(history.astype(jnp.int32), pad_h).reshape(-1),
                   time_interval.astype(jnp.int32),
                   time_weekday.astype(jnp.int32),
                   timespan_interval.astype(jnp.int32),
                   product_id.astype(jnp.int32), cold.astype(jnp.int32),
                   emb_table, field_table.reshape(-1), user_table, doc_table,
                   time_table, weekday_table, timespan_table, product_table,
                   cold_table)
    return _mlp(feat.reshape(_B, _FROWS * 128), W1, b1, W2, b2, Wd, bd)


# speedup vs baseline: 1.1307x; 1.1307x over previous
import functools

import jax
import jax.numpy as jnp
from jax import lax
from jax.experimental import pallas as pl
from jax.experimental.pallas import tpu as pltpu
from jax.experimental.pallas import tpu_sc as plsc

_NC = 2
_NS = 16
_NW = _NC * _NS

_HID = 64
_OTHER = 32
_NF = 101
_B = 1024
_L = 100
_HLEN = 50
_FEAT_ONE = _NF * 2 * _HID
_DIM = _FEAT_ONE + 8 * _OTHER
_FROWS = 104
_LP = 104
_HLP = 56
_RPW = _B // _NW


def _sc_feat_kernel(indexes, values, fields, uid, did, history, ti, wk, ts,
                    pid, cold, emb_table, field_table, user_table, doc_table,
                    time_table, weekday_table, timespan_table, product_table,
                    cold_table, feat,
                    idx_v, fld_v, val_v, hist_v, uid_v, did_v, ti_v, wk_v,
                    ts_v, pid_v, cold_v, emb_rows, ftab_v, pooled, hist_rows,
                    u_rows, d_rows, t_rows, w_rows, s_rows, p_rows, c_rows,
                    sem, sem_g0, sem_g1, sem_w0, sem_w1):
    sems_g = (sem_g0, sem_g1)
    sems_w = (sem_w0, sem_w1)
    wid = lax.axis_index("s") * _NC + lax.axis_index("c")
    base = wid * _RPW

    pltpu.sync_copy(indexes.at[pl.ds(base * _LP, _RPW * _LP)], idx_v)
    pltpu.sync_copy(fields.at[pl.ds(base * _LP, _RPW * _LP)],
                    fld_v.at[pl.ds(0, _RPW * _LP)])
    pltpu.sync_copy(values.at[pl.ds(base * _LP, _RPW * _LP)],
                    val_v.at[pl.ds(0, _RPW * _LP)])
    pltpu.sync_copy(history.at[pl.ds(base * _HLP, _RPW * _HLP)],
                    hist_v.at[pl.ds(0, _RPW * _HLP)])
    sl = pl.ds(base, _RPW)
    pltpu.sync_copy(uid.at[sl], uid_v)
    pltpu.sync_copy(did.at[sl], did_v)
    pltpu.sync_copy(ti.at[sl], ti_v)
    pltpu.sync_copy(wk.at[sl], wk_v)
    pltpu.sync_copy(ts.at[sl], ts_v)
    pltpu.sync_copy(pid.at[sl], pid_v)
    pltpu.sync_copy(cold.at[sl], cold_v)
    pltpu.sync_copy(field_table, ftab_v)

    pltpu.async_copy(user_table.at[uid_v], u_rows, sem).wait()
    pltpu.async_copy(doc_table.at[did_v], d_rows, sem).wait()
    pltpu.async_copy(time_table.at[ti_v], t_rows, sem).wait()
    pltpu.async_copy(weekday_table.at[wk_v], w_rows, sem).wait()
    pltpu.async_copy(timespan_table.at[ts_v], s_rows, sem).wait()
    pltpu.async_copy(product_table.at[pid_v], p_rows, sem).wait()
    pltpu.async_copy(cold_table.at[cold_v], c_rows, sem).wait()

    def fire_gathers(r, p):
        pltpu.async_copy(emb_table.at[idx_v.at[pl.ds(r * _LP, _L)]],
                         emb_rows.at[p], sems_g[p])
        pltpu.async_copy(doc_table.at[hist_v.at[pl.ds(r * _HLP, _HLEN)]],
                         hist_rows.at[p], sems_g[p])

    def compute_row(r, p, do_wait_wb):
        @pl.when(do_wait_wb)
        def _wait_wb():
            pltpu.make_async_copy(pooled.at[p], feat.at[base + r - 2],
                                  sems_w[p]).wait()

        def zero_body(i, c):
            for cc in range(8):
                pooled[p, i, pl.ds(cc * 16, 16)] = jnp.zeros((16,),
                                                             jnp.float32)
            return c
        lax.fori_loop(0, _NF, zero_body, 0)

        pltpu.make_async_copy(emb_table.at[idx_v.at[pl.ds(r * _LP, _L)]],
                              emb_rows.at[p], sems_g[p]).wait()
        pltpu.make_async_copy(doc_table.at[hist_v.at[pl.ds(r * _HLP, _HLEN)]],
                              hist_rows.at[p], sems_g[p]).wait()

        def acc_lanes(lc, nlanes):
            vv = val_v[pl.ds(r * _LP + lc * 16, 16)]
            fv = fld_v[pl.ds(r * _LP + lc * 16, 16)]
            for j in range(nlanes):
                v = vv[j]
                f = fv[j]
                l = lc * 16 + j
                for cc in range(_HID // 16):
                    e = emb_rows[p, l, pl.ds(cc * 16, 16)]
                    plsc.addupdate(pooled.at[p, f, pl.ds(cc * 16, 16)], e * v)
                for cc in range(_HID // 16):
                    ft = ftab_v[pl.ds(f * _HID + cc * 16, 16)]
                    plsc.addupdate(pooled.at[p, f, pl.ds(_HID + cc * 16, 16)],
                                   ft * v)

        def acc_body(lc, c):
            acc_lanes(lc, 16)
            return c
        lax.fori_loop(0, _L // 16, acc_body, 0)
        acc_lanes(_L // 16, _L % 16)

        z16 = jnp.zeros((16,), jnp.float32)

        def hist_lanes(lc, nlanes, carry):
            s0, s1, cnt = carry
            hv = hist_v[pl.ds(r * _HLP + lc * 16, 16)]
            for j in range(nlanes):
                l = lc * 16 + j
                m = jnp.where(hv[j] != 0, jnp.float32(1.0), jnp.float32(0.0))
                s0 = s0 + hist_rows[p, l, pl.ds(0, 16)] * m
                s1 = s1 + hist_rows[p, l, pl.ds(16, 16)] * m
                cnt = cnt + m
            return (s0, s1, cnt)

        def hist_body(lc, carry):
            return hist_lanes(lc, 16, carry)
        carry = lax.fori_loop(0, _HLEN // 16, hist_body,
                              (z16, z16, jnp.float32(0.0)))
        s0, s1, cnt = hist_lanes(_HLEN // 16, _HLEN % 16, carry)
        inv = (jnp.ones((16,), jnp.float32)
               / jnp.full((16,), jnp.maximum(cnt, 1.0)))

        pooled[p, _NF, pl.ds(64, 16)] = s0 * inv
        pooled[p, _NF, pl.ds(80, 16)] = s1 * inv
        for buf, row, off in ((u_rows, _NF, 0), (d_rows, _NF, 32),
                              (t_rows, _NF, 96), (w_rows, _NF + 1, 0),
                              (s_rows, _NF + 1, 32), (p_rows, _NF + 1, 64),
                              (c_rows, _NF + 1, 96)):
            pooled[p, row, pl.ds(off, 16)] = buf[r, pl.ds(0, 16)]
            pooled[p, row, pl.ds(off + 16, 16)] = buf[r, pl.ds(16, 16)]

        pltpu.async_copy(pooled.at[p], feat.at[base + r], sems_w[p])

    fire_gathers(0, 0)

    def pair_body(g, c):
        fire_gathers(2 * g + 1, 1)
        compute_row(2 * g, 0, g > 0)

        @pl.when(g < _RPW // 2 - 1)
        def _prefetch_even():
            fire_gathers(2 * g + 2, 0)

        compute_row(2 * g + 1, 1, g > 0)
        return c

    lax.fori_loop(0, _RPW // 2, pair_body, 0)
    pltpu.make_async_copy(pooled.at[0], feat.at[base + _RPW - 2],
                          sems_w[0]).wait()
    pltpu.make_async_copy(pooled.at[1], feat.at[base + _RPW - 1],
                          sems_w[1]).wait()


def _make_sc_feat():
    mesh = plsc.VectorSubcoreMesh(core_axis_name="c", subcore_axis_name="s")
    return functools.partial(
        pl.kernel, mesh=mesh,
        compiler_params=pltpu.CompilerParams(use_tc_tiling_on_sc=False),
        out_type=jax.ShapeDtypeStruct((_B, _FROWS, 128), jnp.float32),
        scratch_types=[
            pltpu.VMEM((_RPW * _LP,), jnp.int32),
            pltpu.VMEM((_RPW * _LP + 16,), jnp.int32),
            pltpu.VMEM((_RPW * _LP + 16,), jnp.float32),
            pltpu.VMEM((_RPW * _HLP + 16,), jnp.int32),
            pltpu.VMEM((_RPW,), jnp.int32),
            pltpu.VMEM((_RPW,), jnp.int32),
            pltpu.VMEM((_RPW,), jnp.int32),
            pltpu.VMEM((_RPW,), jnp.int32),
            pltpu.VMEM((_RPW,), jnp.int32),
            pltpu.VMEM((_RPW,), jnp.int32),
            pltpu.VMEM((_RPW,), jnp.int32),
            pltpu.VMEM((2, _L, _HID), jnp.float32),
            pltpu.VMEM((_NF * _HID,), jnp.float32),
            pltpu.VMEM((2, _FROWS, 128), jnp.float32),
            pltpu.VMEM((2, _HLEN, _OTHER), jnp.float32),
            pltpu.VMEM((_RPW, _OTHER), jnp.float32),
            pltpu.VMEM((_RPW, _OTHER), jnp.float32),
            pltpu.VMEM((_RPW, _OTHER), jnp.float32),
            pltpu.VMEM((_RPW, _OTHER), jnp.float32),
            pltpu.VMEM((_RPW, _OTHER), jnp.float32),
            pltpu.VMEM((_RPW, _OTHER), jnp.float32),
            pltpu.VMEM((_RPW, _OTHER), jnp.float32),
            pltpu.SemaphoreType.DMA,
            pltpu.SemaphoreType.DMA,
            pltpu.SemaphoreType.DMA,
            pltpu.SemaphoreType.DMA,
            pltpu.SemaphoreType.DMA,
        ])(_sc_feat_kernel)


def _mlp_body(feat_ref, w1_ref, b1_ref, w2_ref, b2_ref, wd_ref, bd_ref,
              out_ref):
    x = feat_ref[...].reshape(-1, _FROWS * 128)[:, :_DIM].astype(jnp.bfloat16)
    h = jnp.dot(x, w1_ref[...], preferred_element_type=jnp.float32)
    h = jnp.maximum(h + b1_ref[...], 0.0)
    h = jnp.dot(h.astype(jnp.bfloat16), w2_ref[...],
                preferred_element_type=jnp.float32)
    h = jnp.maximum(h + b2_ref[...], 0.0)
    out_ref[...] = (jnp.dot(h, wd_ref[...], preferred_element_type=jnp.float32)
                    + bd_ref[...])


def _mlp(feat, W1, b1, W2, b2, Wd, bd):
    mb = 128
    grid = (_B // mb,)
    return pl.pallas_call(
        _mlp_body,
        grid=grid,
        in_specs=[
            pl.BlockSpec((mb, _FROWS, 128), lambda i: (i, 0, 0)),
            pl.BlockSpec((_DIM, W1.shape[1]), lambda i: (0, 0)),
            pl.BlockSpec((1, W1.shape[1]), lambda i: (0, 0)),
            pl.BlockSpec((W2.shape[0], W2.shape[1]), lambda i: (0, 0)),
            pl.BlockSpec((1, W2.shape[1]), lambda i: (0, 0)),
            pl.BlockSpec((Wd.shape[0], 1), lambda i: (0, 0)),
            pl.BlockSpec((1, 1), lambda i: (0, 0)),
        ],
        out_specs=pl.BlockSpec((mb, 1), lambda i: (i, 0)),
        out_shape=jax.ShapeDtypeStruct((_B, 1), jnp.float32),
        compiler_params=pltpu.CompilerParams(
            dimension_semantics=("arbitrary",)),
    )(feat, W1.astype(jnp.bfloat16), b1.reshape(1, -1),
      W2.astype(jnp.bfloat16), b2.reshape(1, -1), Wd, bd.reshape(1, 1))


def kernel(indexes, values, fields, uid, did, history, time_interval,
           time_weekday, timespan_interval, product_id, cold, emb_table,
           field_table, user_table, doc_table, time_table, weekday_table,
           timespan_table, product_table, cold_table, W1, b1, W2, b2, Wd, bd):
    sc_feat = _make_sc_feat()
    pad_l = ((0, 0), (0, _LP - _L))
    pad_h = ((0, 0), (0, _HLP - _HLEN))
    feat = sc_feat(jnp.pad(indexes.astype(jnp.int32), pad_l).reshape(-1),
                   jnp.pad(values, pad_l).reshape(-1),
                   jnp.pad(fields.astype(jnp.int32), pad_l).reshape(-1),
                   uid.astype(jnp.int32), did.astype(jnp.int32),
                   jnp.pad(history.astype(jnp.int32), pad_h).reshape(-1),
                   time_interval.astype(jnp.int32),
                   time_weekday.astype(jnp.int32),
                   timespan_interval.astype(jnp.int32),
                   product_id.astype(jnp.int32), cold.astype(jnp.int32),
                   emb_table, field_table.reshape(-1), user_table, doc_table,
                   time_table, weekday_table, timespan_table, product_table,
                   cold_table)
    return _mlp(feat, W1, b1, W2, b2, Wd, bd)
